# baseline (speedup 1.0000x reference)
import functools

import jax
import jax.numpy as jnp
from jax import lax
from jax.experimental import pallas as pl
from jax.experimental.pallas import tpu as pltpu
from jax.experimental.pallas import tpu_sc as plsc

VOCAB = 1000000
EMBED_DIM = 64
BATCH = 16384

_INFO = plsc.get_sparse_core_info()
_NC = _INFO.num_cores
_NS = _INFO.num_subcores
_NW = _NC * _NS
_BPW = BATCH // _NW

_C = 256
_SH = 8
_NFULL = VOCAB // _C
_TAIL = VOCAB - _NFULL * _C
_TAIL_CID = _NFULL
_KMAX = 244
_NSK = 8
_RING = 4
_CAP = 64
_ROWW = 128
_DUMP0 = BATCH

_COMPILER_PARAMS = pltpu.CompilerParams(
    needs_layout_passes=False, use_tc_tiling_on_sc=True)


def _extract_body(idx_hbm, tab_hbm, tail_hbm, out_hbm,
                  idxbuf, hits_b, sup_b, cbufs, tailbuf, packed, bid2d,
                  sems, semS, tile):
    lanes = lax.iota(jnp.int32, 16)
    ones = jnp.ones((16,), jnp.int32)
    dump = jnp.zeros((16,), jnp.int32) + (_DUMP0 + tile)

    def start(k, buf, sem):
        nchunks = _KMAX + jnp.where(tile < 2, 1, 0)

        @pl.when(k < nchunks)
        def _():
            cid = tile + 16 * k
            pltpu.async_copy(tab_hbm.at[:, pl.ds(cid * _C, _C)], buf, sem)

    for p in range(_RING):
        start(jnp.int32(p), cbufs[p], sems[p])

    pltpu.sync_copy(idx_hbm, idxbuf)

    def filt(i, n):
        v = idxbuf[pl.ds(i * 16, 16)]
        m = ((v >> _SH) & 15) == tile
        plsc.store_compressed(hits_b.at[pl.ds(n, 16)], lanes + i * 16, mask=m)
        return n + plsc.all_reduce_population_count(m)[0]

    n = lax.fori_loop(0, BATCH // 16, filt, jnp.int32(0))

    sup_off = [jnp.int32(0)]
    supv = jnp.zeros((16,), jnp.int32)
    off = jnp.int32(0)
    for sk in range(_NSK):
        def bucket(i, off, sk=sk):
            b = hits_b[pl.ds(i * 16, 16)] & (BATCH - 1)
            v = plsc.load_gather(idxbuf, [b])
            m = ((v >> 17) == sk) & ((i * 16 + lanes) < n)
            plsc.store_compressed(sup_b.at[pl.ds(off, 16)], b, mask=m)
            return off + plsc.all_reduce_population_count(m)[0]

        off = lax.fori_loop(0, (n + 15) // 16, bucket, off)
        sup_off.append(off)
        supv = jnp.where(lanes == (sk + 1), off, supv)

    for q in range(_CAP // 16):
        bid2d[0, pl.ds(q * 16, 16)] = dump

    def flush_if(pred, n_pk):
        @pl.when(pred)
        def _():
            pltpu.async_copy(packed, out_hbm.at[bid2d.at[0]], semS).wait()
        return jnp.where(pred, 0, n_pk)

    def walk_chunk(cid, n_pk, lo, hi, buf):
        def w(i, n_pk):
            p0 = lo + i * 16
            b = sup_b[pl.ds(p0, 16)] & (BATCH - 1)
            v = plsc.load_gather(idxbuf, [b])
            m = ((v >> _SH) == cid) & ((p0 + lanes) < hi)
            cnt = plsc.all_reduce_population_count(m)[0]

            @pl.when(cnt > 0)
            def _():
                vloc = v & (_C - 1)
                pos = n_pk + plsc.cumsum(jnp.where(m, ones, 0), mask=m) - 1

                def dstep(d8, carry):
                    for dd in range(8):
                        d = d8 * 8 + dd
                        if buf is None:
                            vals = plsc.load_gather(
                                tailbuf, [(vloc & (_TAIL - 1)) + d * _TAIL],
                                mask=m)
                        else:
                            vals = plsc.load_gather(
                                buf, [jnp.zeros((16,), jnp.int32) + d, vloc],
                                mask=m)
                        plsc.store_scatter(
                            packed, [pos, jnp.zeros((16,), jnp.int32) + d],
                            vals, mask=m)
                    return carry

                lax.fori_loop(0, EMBED_DIM // 8, dstep, 0)
                plsc.store_scatter(bid2d, [jnp.zeros((16,), jnp.int32), pos],
                                   b, mask=m)

            n_pk = n_pk + cnt
            return flush_if(n_pk > _CAP - 16, n_pk)

        return lax.fori_loop(0, (hi - lo + 15) // 16, w, n_pk)

    def ring_group(j4, n_pk):
        k0 = _RING * j4
        for p in range(_RING):
            k = k0 + p
            sk = k >> 5
            lo = supv.at[jnp.zeros((16,), jnp.int32) + sk].get(
                mode="promise_in_bounds")[0]
            hi = supv.at[jnp.zeros((16,), jnp.int32) + (sk + 1)].get(
                mode="promise_in_bounds")[0]
            pltpu.make_async_copy(
                tab_hbm.at[:, pl.ds(0, _C)], cbufs[p], sems[p]).wait()
            n_pk = walk_chunk(tile + 16 * k, n_pk, lo, hi, cbufs[p])
            start(k + _RING, cbufs[p], sems[p])
        return n_pk

    n_pk = lax.fori_loop(0, _KMAX // _RING, ring_group, jnp.int32(0))

    @pl.when(tile < 2)
    def _():
        pltpu.make_async_copy(tab_hbm.at[:, pl.ds(0, _C)], cbufs[0],
                              sems[0]).wait()

    n_pk = jnp.where(
        tile < 2,
        walk_chunk(jnp.int32(16 * _KMAX) + tile, n_pk, sup_off[7],
                   sup_off[8], cbufs[0]),
        n_pk)

    @pl.when(tile == (_TAIL_CID % 16))
    def _():
        pltpu.sync_copy(tail_hbm, tailbuf)

    n_pk = walk_chunk(jnp.int32(_TAIL_CID), n_pk, sup_off[7], sup_off[8],
                      None)
    flush_if(n_pk > 0, n_pk)


def _extract_kernel_body(c_hbm, w_hbm, ct_hbm, wt_hbm, tailc_hbm, tailw_hbm,
                         cv_hbm, wv_hbm,
                         idxbuf, hits_b, sup_b, cbuf0, cbuf1, cbuf2, cbuf3,
                         tailbuf, packed, bid2d, sem0, sem1, sem2, sem3,
                         semS):
    core = lax.axis_index("c")
    tile = lax.axis_index("s")
    cbufs = (cbuf0, cbuf1, cbuf2, cbuf3)
    sems = (sem0, sem1, sem2, sem3)

    @pl.when(core == 0)
    def _():
        _extract_body(c_hbm, ct_hbm, tailc_hbm, cv_hbm,
                      idxbuf, hits_b, sup_b, cbufs, tailbuf, packed,
                      bid2d, sems, semS, tile)

    @pl.when(core == 1)
    def _():
        _extract_body(w_hbm, wt_hbm, tailw_hbm, wv_hbm,
                      idxbuf, hits_b, sup_b, cbufs, tailbuf, packed,
                      bid2d, sems, semS, tile)


def _dot_body(cv_hbm, wv_hbm, out_hbm, cvb, wvb, pscr, out_v, sem):
    wid = lax.axis_index("s") * _NC + lax.axis_index("c")
    base = wid * _BPW
    half = _BPW // 2
    lanes = lax.iota(jnp.int32, 16)

    for h in range(2):
        hbase = base + h * half
        c1 = pltpu.async_copy(cv_hbm.at[pl.ds(hbase, half), :], cvb, sem)
        c2 = pltpu.async_copy(wv_hbm.at[pl.ds(hbase, half), :], wvb, sem)
        c1.wait()
        c2.wait()

        def group(g, carry):
            rbase = g * 16
            for r in range(16):
                row = rbase + r
                acc = cvb[row, pl.ds(0, 16)] * wvb[row, pl.ds(0, 16)]
                for k in range(1, EMBED_DIM // 16):
                    acc = acc + (cvb[row, pl.ds(k * 16, 16)]
                                 * wvb[row, pl.ds(k * 16, 16)])
                pscr[r, :] = acc
            tot = plsc.load_gather(pscr, [lanes, jnp.zeros((16,), jnp.int32)])
            for j in range(1, 16):
                tot = tot + plsc.load_gather(
                    pscr, [lanes, jnp.zeros((16,), jnp.int32) + j])
            out_v[pl.ds(h * half + rbase, 16)] = 1.0 / (1.0 + jnp.exp(-tot))
            return carry

        lax.fori_loop(0, half // 16, group, 0)

    pltpu.sync_copy(out_v, out_hbm.at[pl.ds(base, _BPW)])


@jax.jit
def _sgns(c, w, ct, wt, tailc, tailw):
    mesh = plsc.VectorSubcoreMesh(core_axis_name="c", subcore_axis_name="s")
    extract = functools.partial(
        pl.kernel,
        mesh=mesh,
        compiler_params=_COMPILER_PARAMS,
        out_type=(jax.ShapeDtypeStruct((BATCH + 16, _ROWW), jnp.float32),
                  jax.ShapeDtypeStruct((BATCH + 16, _ROWW), jnp.float32)),
        scratch_types=[
            pltpu.VMEM((BATCH,), jnp.int32),
            pltpu.VMEM((BATCH + 16,), jnp.int32),
            pltpu.VMEM((BATCH + 16,), jnp.int32),
            pltpu.VMEM((EMBED_DIM, _C), jnp.float32),
            pltpu.VMEM((EMBED_DIM, _C), jnp.float32),
            pltpu.VMEM((EMBED_DIM, _C), jnp.float32),
            pltpu.VMEM((EMBED_DIM, _C), jnp.float32),
            pltpu.VMEM((EMBED_DIM * _TAIL,), jnp.float32),
            pltpu.VMEM((_CAP, _ROWW), jnp.float32),
            pltpu.VMEM((1, _CAP), jnp.int32),
            pltpu.SemaphoreType.DMA,
            pltpu.SemaphoreType.DMA,
            pltpu.SemaphoreType.DMA,
            pltpu.SemaphoreType.DMA,
            pltpu.SemaphoreType.DMA,
        ],
    )(_extract_kernel_body)
    cv, wv = extract(c, w, ct, wt, tailc, tailw)

    dot = functools.partial(
        pl.kernel,
        mesh=mesh,
        compiler_params=_COMPILER_PARAMS,
        out_type=jax.ShapeDtypeStruct((BATCH,), jnp.float32),
        scratch_types=[
            pltpu.VMEM((_BPW // 2, _ROWW), jnp.float32),
            pltpu.VMEM((_BPW // 2, _ROWW), jnp.float32),
            pltpu.VMEM((16, 16), jnp.float32),
            pltpu.VMEM((_BPW,), jnp.float32),
            pltpu.SemaphoreType.DMA,
        ],
    )(_dot_body)
    return dot(cv, wv)


def kernel(c, w, c_embeds, w_embeds):
    tailc = c_embeds[_NFULL * _C:, :].T.reshape(-1)
    tailw = w_embeds[_NFULL * _C:, :].T.reshape(-1)
    return _sgns(c.astype(jnp.int32), w.astype(jnp.int32),
                 c_embeds.T, w_embeds.T, tailc, tailw)


# speedup vs baseline: 1.1753x; 1.1753x over previous
import functools

import jax
import jax.numpy as jnp
from jax import lax
from jax.experimental import pallas as pl
from jax.experimental.pallas import tpu as pltpu
from jax.experimental.pallas import tpu_sc as plsc

VOCAB = 1000000
EMBED_DIM = 64
BATCH = 16384

_INFO = plsc.get_sparse_core_info()
_NC = _INFO.num_cores
_NS = _INFO.num_subcores
_NW = _NC * _NS
_BPW = BATCH // _NW

_C = 512
_NFULL = VOCAB // _C
_TAIL = VOCAB - _NFULL * _C
_TAIL_CID = _NFULL
_KMAX = 122
_NSK = 8
_CAP = 64
_ROWW = 128
_DUMP0 = BATCH

_COMPILER_PARAMS = pltpu.CompilerParams(
    needs_layout_passes=False, use_tc_tiling_on_sc=True)


def _extract_body(idx_hbm, tab_hbm, tail_hbm, out_hbm,
                  idxbuf, hits_b, sup_b, cbufA, cbufB, tailbuf, packed, bid2d,
                  semA, semB, semS, tile):
    lanes = lax.iota(jnp.int32, 16)
    ones = jnp.ones((16,), jnp.int32)
    dump = jnp.zeros((16,), jnp.int32) + (_DUMP0 + tile)

    def start(k, buf, sem):
        nchunks = _KMAX + jnp.where(tile == 0, 1, 0)

        @pl.when(k < nchunks)
        def _():
            cid = tile + 16 * k
            pltpu.async_copy(tab_hbm.at[:, pl.ds(cid * _C, _C)], buf, sem)

    start(jnp.int32(0), cbufA, semA)
    start(jnp.int32(1), cbufB, semB)

    pltpu.sync_copy(idx_hbm, idxbuf)

    def filt(i, carry):
        n0, n1 = carry
        v0 = idxbuf[pl.ds(i * 16, 16)]
        v1 = idxbuf[pl.ds(BATCH // 2 + i * 16, 16)]
        m0 = ((v0 >> 9) & 15) == tile
        m1 = ((v1 >> 9) & 15) == tile
        plsc.store_compressed(hits_b.at[pl.ds(n0, 16)], lanes + i * 16,
                              mask=m0)
        plsc.store_compressed(
            hits_b.at[pl.ds(BATCH // 2 + 16 + n1, 16)],
            lanes + (BATCH // 2 + i * 16), mask=m1)
        return (n0 + plsc.all_reduce_population_count(m0)[0],
                n1 + plsc.all_reduce_population_count(m1)[0])

    n0, n1 = lax.fori_loop(0, BATCH // 32, filt,
                           (jnp.int32(0), jnp.int32(0)))

    sup_off = [jnp.int32(0)]
    off = jnp.int32(0)
    for sk in range(_NSK):
        def bucket(base, nn, i, off, sk=sk):
            b = hits_b[pl.ds(base + i * 16, 16)] & (BATCH - 1)
            v = plsc.load_gather(idxbuf, [b])
            m = ((v >> 17) == sk) & ((i * 16 + lanes) < nn)
            plsc.store_compressed(sup_b.at[pl.ds(off, 16)], b, mask=m)
            return off + plsc.all_reduce_population_count(m)[0]

        off = lax.fori_loop(
            0, (n0 + 15) // 16,
            functools.partial(bucket, 0, n0), off)
        off = lax.fori_loop(
            0, (n1 + 15) // 16,
            functools.partial(bucket, BATCH // 2 + 16, n1), off)
        sup_off.append(off)

    for q in range(_CAP // 16):
        bid2d[0, pl.ds(q * 16, 16)] = dump

    def flush_if(pred, n_pk):
        @pl.when(pred)
        def _():
            pltpu.async_copy(packed, out_hbm.at[bid2d.at[0]], semS).wait()
        return jnp.where(pred, 0, n_pk)

    def walk_chunk(cid, n_pk, lo, hi, buf):
        def w(i, n_pk):
            p0 = lo + i * 16
            b = sup_b[pl.ds(p0, 16)] & (BATCH - 1)
            v = plsc.load_gather(idxbuf, [b])
            m = ((v >> 9) == cid) & ((p0 + lanes) < hi)
            cnt = plsc.all_reduce_population_count(m)[0]

            @pl.when(cnt > 0)
            def _():
                vloc = v & (_C - 1)
                pos = n_pk + plsc.cumsum(jnp.where(m, ones, 0), mask=m) - 1

                def dstep(d8, carry):
                    for dd in range(8):
                        d = d8 * 8 + dd
                        if buf is None:
                            vals = plsc.load_gather(
                                tailbuf, [(vloc & (_TAIL - 1)) + d * _TAIL],
                                mask=m)
                        else:
                            vals = plsc.load_gather(
                                buf, [jnp.zeros((16,), jnp.int32) + d, vloc],
                                mask=m)
                        plsc.store_scatter(
                            packed, [pos, jnp.zeros((16,), jnp.int32) + d],
                            vals, mask=m)
                    return carry

                lax.fori_loop(0, EMBED_DIM // 8, dstep, 0)
                plsc.store_scatter(bid2d, [jnp.zeros((16,), jnp.int32), pos],
                                   b, mask=m)

            n_pk = n_pk + cnt
            return flush_if(n_pk > _CAP - 16, n_pk)

        return lax.fori_loop(0, (hi - lo + 15) // 16, w, n_pk)

    n_pk = jnp.int32(0)
    for sk in range(_NSK):
        npairs = 8 if sk < _NSK - 1 else 5
        lo = sup_off[sk]
        hi = sup_off[sk + 1]

        def pair(j2, n_pk, sk=sk, lo=lo, hi=hi):
            k = 16 * sk + 2 * j2
            for p, buf, sem in ((0, cbufA, semA), (1, cbufB, semB)):
                pltpu.make_async_copy(
                    tab_hbm.at[:, pl.ds(0, _C)], buf, sem).wait()
                n_pk = walk_chunk(tile + 16 * (k + p), n_pk, lo, hi, buf)
                start(k + p + 2, buf, sem)
            return n_pk

        n_pk = lax.fori_loop(0, npairs, pair, n_pk)

    @pl.when(tile == 0)
    def _():
        pltpu.make_async_copy(tab_hbm.at[:, pl.ds(0, _C)], cbufA, semA).wait()

    n_pk = jnp.where(
        tile == 0,
        walk_chunk(jnp.int32(1952) + tile, n_pk, sup_off[7], sup_off[8],
                   cbufA),
        n_pk)

    @pl.when(tile == (_TAIL_CID % 16))
    def _():
        pltpu.sync_copy(tail_hbm, tailbuf)

    n_pk = walk_chunk(jnp.int32(_TAIL_CID), n_pk, sup_off[7], sup_off[8],
                      None)
    flush_if(n_pk > 0, n_pk)


def _extract_kernel_body(c_hbm, w_hbm, ct_hbm, wt_hbm, tailc_hbm, tailw_hbm,
                         cv_hbm, wv_hbm,
                         idxbuf, hits_b, sup_b, cbufA, cbufB, tailbuf, packed,
                         bid2d, semA, semB, semS):
    core = lax.axis_index("c")
    tile = lax.axis_index("s")

    @pl.when(core == 0)
    def _():
        _extract_body(c_hbm, ct_hbm, tailc_hbm, cv_hbm,
                      idxbuf, hits_b, sup_b, cbufA, cbufB, tailbuf, packed,
                      bid2d, semA, semB, semS, tile)

    @pl.when(core == 1)
    def _():
        _extract_body(w_hbm, wt_hbm, tailw_hbm, wv_hbm,
                      idxbuf, hits_b, sup_b, cbufA, cbufB, tailbuf, packed,
                      bid2d, semA, semB, semS, tile)


def _dot_body(cv_hbm, wv_hbm, out_hbm, cvb, wvb, pscr, out_v, sem):
    wid = lax.axis_index("s") * _NC + lax.axis_index("c")
    base = wid * _BPW
    half = _BPW // 2
    lanes = lax.iota(jnp.int32, 16)

    for h in range(2):
        hbase = base + h * half
        c1 = pltpu.async_copy(cv_hbm.at[pl.ds(hbase, half), :], cvb, sem)
        c2 = pltpu.async_copy(wv_hbm.at[pl.ds(hbase, half), :], wvb, sem)
        c1.wait()
        c2.wait()

        def group(g, carry):
            rbase = g * 16
            for r in range(16):
                row = rbase + r
                acc = cvb[row, pl.ds(0, 16)] * wvb[row, pl.ds(0, 16)]
                for k in range(1, EMBED_DIM // 16):
                    acc = acc + (cvb[row, pl.ds(k * 16, 16)]
                                 * wvb[row, pl.ds(k * 16, 16)])
                pscr[r, :] = acc
            tot = plsc.load_gather(pscr, [lanes, jnp.zeros((16,), jnp.int32)])
            for j in range(1, 16):
                tot = tot + plsc.load_gather(
                    pscr, [lanes, jnp.zeros((16,), jnp.int32) + j])
            out_v[pl.ds(h * half + rbase, 16)] = 1.0 / (1.0 + jnp.exp(-tot))
            return carry

        lax.fori_loop(0, half // 16, group, 0)

    pltpu.sync_copy(out_v, out_hbm.at[pl.ds(base, _BPW)])


@jax.jit
def _sgns(c, w, ct, wt, tailc, tailw):
    mesh = plsc.VectorSubcoreMesh(core_axis_name="c", subcore_axis_name="s")
    extract = functools.partial(
        pl.kernel,
        mesh=mesh,
        compiler_params=_COMPILER_PARAMS,
        out_type=(jax.ShapeDtypeStruct((BATCH + 16, _ROWW), jnp.float32),
                  jax.ShapeDtypeStruct((BATCH + 16, _ROWW), jnp.float32)),
        scratch_types=[
            pltpu.VMEM((BATCH,), jnp.int32),
            pltpu.VMEM((BATCH + 32,), jnp.int32),
            pltpu.VMEM((BATCH + 16,), jnp.int32),
            pltpu.VMEM((EMBED_DIM, _C), jnp.float32),
            pltpu.VMEM((EMBED_DIM, _C), jnp.float32),
            pltpu.VMEM((EMBED_DIM * _TAIL,), jnp.float32),
            pltpu.VMEM((_CAP, _ROWW), jnp.float32),
            pltpu.VMEM((1, _CAP), jnp.int32),
            pltpu.SemaphoreType.DMA,
            pltpu.SemaphoreType.DMA,
            pltpu.SemaphoreType.DMA,
        ],
    )(_extract_kernel_body)
    cv, wv = extract(c, w, ct, wt, tailc, tailw)

    dot = functools.partial(
        pl.kernel,
        mesh=mesh,
        compiler_params=_COMPILER_PARAMS,
        out_type=jax.ShapeDtypeStruct((BATCH,), jnp.float32),
        scratch_types=[
            pltpu.VMEM((_BPW // 2, _ROWW), jnp.float32),
            pltpu.VMEM((_BPW // 2, _ROWW), jnp.float32),
            pltpu.VMEM((16, 16), jnp.float32),
            pltpu.VMEM((_BPW,), jnp.float32),
            pltpu.SemaphoreType.DMA,
        ],
    )(_dot_body)
    return dot(cv, wv)


def kernel(c, w, c_embeds, w_embeds):
    tailc = c_embeds[_NFULL * _C:, :].T.reshape(-1)
    tailw = w_embeds[_NFULL * _C:, :].T.reshape(-1)
    return _sgns(c.astype(jnp.int32), w.astype(jnp.int32),
                 c_embeds.T, w_embeds.T, tailc, tailw)


# speedup vs baseline: 1.1798x; 1.0039x over previous
import functools

import jax
import jax.numpy as jnp
from jax import lax
from jax.experimental import pallas as pl
from jax.experimental.pallas import tpu as pltpu
from jax.experimental.pallas import tpu_sc as plsc

VOCAB = 1000000
EMBED_DIM = 64
BATCH = 16384

_INFO = plsc.get_sparse_core_info()
_NC = _INFO.num_cores
_NS = _INFO.num_subcores
_NW = _NC * _NS
_BPW = BATCH // _NW

_C = 512
_NFULL = VOCAB // _C
_TAIL = VOCAB - _NFULL * _C
_TAIL_CID = _NFULL
_KMAX = 122
_NSK = 8
_CAP = 64
_ROWW = 128
_DUMP0 = BATCH

_COMPILER_PARAMS = pltpu.CompilerParams(
    needs_layout_passes=False, use_tc_tiling_on_sc=True)


def _extract_body(idx_hbm, tab_hbm, tail_hbm, out_hbm,
                  idxbuf, hits_b, sup_b, cbufA, cbufB, tailbuf, packed, bid2d,
                  semA, semB, semS, tile):
    lanes = lax.iota(jnp.int32, 16)
    ones = jnp.ones((16,), jnp.int32)
    dump = jnp.zeros((16,), jnp.int32) + (_DUMP0 + tile)

    def start(k, buf, sem):
        nchunks = _KMAX + jnp.where(tile == 0, 1, 0)

        @pl.when(k < nchunks)
        def _():
            cid = tile + 16 * k
            pltpu.async_copy(tab_hbm.at[:, pl.ds(cid * _C, _C)], buf, sem)

    start(jnp.int32(0), cbufA, semA)
    start(jnp.int32(1), cbufB, semB)

    pltpu.sync_copy(idx_hbm, idxbuf)

    def filt(i, carry):
        n0, n1 = carry
        v0 = idxbuf[pl.ds(i * 16, 16)]
        v1 = idxbuf[pl.ds(BATCH // 2 + i * 16, 16)]
        m0 = ((v0 >> 9) & 15) == tile
        m1 = ((v1 >> 9) & 15) == tile
        plsc.store_compressed(hits_b.at[pl.ds(n0, 16)], lanes + i * 16,
                              mask=m0)
        plsc.store_compressed(
            hits_b.at[pl.ds(BATCH // 2 + 16 + n1, 16)],
            lanes + (BATCH // 2 + i * 16), mask=m1)
        return (n0 + plsc.all_reduce_population_count(m0)[0],
                n1 + plsc.all_reduce_population_count(m1)[0])

    n0, n1 = lax.fori_loop(0, BATCH // 32, filt,
                           (jnp.int32(0), jnp.int32(0)))

    sup_off = [jnp.int32(0)]
    off = jnp.int32(0)
    for sk in range(_NSK):
        def bucket(base, nn, i, off, sk=sk):
            b = hits_b[pl.ds(base + i * 16, 16)] & (BATCH - 1)
            v = plsc.load_gather(idxbuf, [b])
            m = ((v >> 17) == sk) & ((i * 16 + lanes) < nn)
            plsc.store_compressed(sup_b.at[pl.ds(off, 16)], b, mask=m)
            return off + plsc.all_reduce_population_count(m)[0]

        off = lax.fori_loop(
            0, (n0 + 15) // 16,
            functools.partial(bucket, 0, n0), off)
        off = lax.fori_loop(
            0, (n1 + 15) // 16,
            functools.partial(bucket, BATCH // 2 + 16, n1), off)
        sup_off.append(off)

    for q in range(_CAP // 16):
        bid2d[0, pl.ds(q * 16, 16)] = dump

    def flush_if(pred, n_pk):
        @pl.when(pred)
        def _():
            pltpu.async_copy(packed, out_hbm.at[bid2d.at[0]], semS).wait()
        return jnp.where(pred, 0, n_pk)

    def walk_chunk(cid, n_pk, lo, hi, buf):
        def w(i, n_pk):
            p0 = lo + i * 16
            b = sup_b[pl.ds(p0, 16)] & (BATCH - 1)
            v = plsc.load_gather(idxbuf, [b])
            m = ((v >> 9) == cid) & ((p0 + lanes) < hi)
            cnt = plsc.all_reduce_population_count(m)[0]

            @pl.when(cnt > 0)
            def _():
                vloc = v & (_C - 1)
                pos = n_pk + plsc.cumsum(jnp.where(m, ones, 0), mask=m) - 1

                def dstep(d8, carry):
                    for dd in range(8):
                        d = d8 * 8 + dd
                        if buf is None:
                            vals = plsc.load_gather(
                                tailbuf, [(vloc & (_TAIL - 1)) + d * _TAIL],
                                mask=m)
                        else:
                            vals = plsc.load_gather(
                                buf, [jnp.zeros((16,), jnp.int32) + d, vloc],
                                mask=m)
                        plsc.store_scatter(
                            packed, [pos, jnp.zeros((16,), jnp.int32) + d],
                            vals, mask=m)
                    return carry

                lax.fori_loop(0, EMBED_DIM // 8, dstep, 0)
                plsc.store_scatter(bid2d, [jnp.zeros((16,), jnp.int32), pos],
                                   b, mask=m)

            n_pk = n_pk + cnt
            return flush_if(n_pk > _CAP - 16, n_pk)

        return lax.fori_loop(0, (hi - lo + 15) // 16, w, n_pk)

    n_pk = jnp.int32(0)
    for sk in range(_NSK):
        npairs = 8 if sk < _NSK - 1 else 5
        lo = sup_off[sk]
        hi = sup_off[sk + 1]

        def pair(j2, n_pk, sk=sk, lo=lo, hi=hi):
            k = 16 * sk + 2 * j2
            for p, buf, sem in ((0, cbufA, semA), (1, cbufB, semB)):
                pltpu.make_async_copy(
                    tab_hbm.at[:, pl.ds(0, _C)], buf, sem).wait()
                n_pk = walk_chunk(tile + 16 * (k + p), n_pk, lo, hi, buf)
                start(k + p + 2, buf, sem)
            return n_pk

        n_pk = lax.fori_loop(0, npairs, pair, n_pk)

    @pl.when(tile == 0)
    def _():
        pltpu.make_async_copy(tab_hbm.at[:, pl.ds(0, _C)], cbufA, semA).wait()

    n_pk = jnp.where(
        tile == 0,
        walk_chunk(jnp.int32(1952) + tile, n_pk, sup_off[7], sup_off[8],
                   cbufA),
        n_pk)

    @pl.when(tile == (_TAIL_CID % 16))
    def _():
        pltpu.sync_copy(tail_hbm, tailbuf)

    n_pk = walk_chunk(jnp.int32(_TAIL_CID), n_pk, sup_off[7], sup_off[8],
                      None)
    flush_if(n_pk > 0, n_pk)


def _extract_kernel_body(c_hbm, w_hbm, ct_hbm, wt_hbm, tailc_hbm, tailw_hbm,
                         cv_hbm, wv_hbm,
                         idxbuf, hits_b, sup_b, cbufA, cbufB, tailbuf, packed,
                         bid2d, semA, semB, semS):
    core = lax.axis_index("c")
    tile = lax.axis_index("s")

    @pl.when(core == 0)
    def _():
        _extract_body(c_hbm, ct_hbm, tailc_hbm, cv_hbm,
                      idxbuf, hits_b, sup_b, cbufA, cbufB, tailbuf, packed,
                      bid2d, semA, semB, semS, tile)

    @pl.when(core == 1)
    def _():
        _extract_body(w_hbm, wt_hbm, tailw_hbm, wv_hbm,
                      idxbuf, hits_b, sup_b, cbufA, cbufB, tailbuf, packed,
                      bid2d, semA, semB, semS, tile)


def _dot_body(cv_hbm, wv_hbm, out_hbm, cvb0, wvb0, cvb1, wvb1, pscr, out_v,
              sem0, sem1):
    wid = lax.axis_index("s") * _NC + lax.axis_index("c")
    base = wid * _BPW
    quarter = _BPW // 4
    lanes = lax.iota(jnp.int32, 16)
    bufs = ((cvb0, wvb0, sem0), (cvb1, wvb1, sem1))

    def fetch(q, cvb, wvb, sem):
        qbase = base + q * quarter
        pltpu.async_copy(cv_hbm.at[pl.ds(qbase, quarter), :], cvb, sem)
        pltpu.async_copy(wv_hbm.at[pl.ds(qbase, quarter), :], wvb, sem)

    for p in range(2):
        fetch(p, *bufs[p])

    for q in range(4):
        cvb, wvb, sem = bufs[q % 2]
        pltpu.make_async_copy(cv_hbm.at[pl.ds(0, quarter), :], cvb,
                              sem).wait()
        pltpu.make_async_copy(wv_hbm.at[pl.ds(0, quarter), :], wvb,
                              sem).wait()

        def group(g, carry, cvb=cvb, wvb=wvb, q=q):
            rbase = g * 16
            for r in range(16):
                row = rbase + r
                acc = cvb[row, pl.ds(0, 16)] * wvb[row, pl.ds(0, 16)]
                for k in range(1, EMBED_DIM // 16):
                    acc = acc + (cvb[row, pl.ds(k * 16, 16)]
                                 * wvb[row, pl.ds(k * 16, 16)])
                pscr[r, :] = acc
            tot = plsc.load_gather(pscr, [lanes, jnp.zeros((16,), jnp.int32)])
            for j in range(1, 16):
                tot = tot + plsc.load_gather(
                    pscr, [lanes, jnp.zeros((16,), jnp.int32) + j])
            out_v[pl.ds(q * quarter + rbase, 16)] = 1.0 / (1.0 + jnp.exp(-tot))
            return carry

        lax.fori_loop(0, quarter // 16, group, 0)
        if q < 2:
            fetch(q + 2, *bufs[q % 2])

    pltpu.sync_copy(out_v, out_hbm.at[pl.ds(base, _BPW)])


@jax.jit
def _sgns(c, w, ct, wt, tailc, tailw):
    mesh = plsc.VectorSubcoreMesh(core_axis_name="c", subcore_axis_name="s")
    extract = functools.partial(
        pl.kernel,
        mesh=mesh,
        compiler_params=_COMPILER_PARAMS,
        out_type=(jax.ShapeDtypeStruct((BATCH + 16, _ROWW), jnp.float32),
                  jax.ShapeDtypeStruct((BATCH + 16, _ROWW), jnp.float32)),
        scratch_types=[
            pltpu.VMEM((BATCH,), jnp.int32),
            pltpu.VMEM((BATCH + 32,), jnp.int32),
            pltpu.VMEM((BATCH + 16,), jnp.int32),
            pltpu.VMEM((EMBED_DIM, _C), jnp.float32),
            pltpu.VMEM((EMBED_DIM, _C), jnp.float32),
            pltpu.VMEM((EMBED_DIM * _TAIL,), jnp.float32),
            pltpu.VMEM((_CAP, _ROWW), jnp.float32),
            pltpu.VMEM((1, _CAP), jnp.int32),
            pltpu.SemaphoreType.DMA,
            pltpu.SemaphoreType.DMA,
            pltpu.SemaphoreType.DMA,
        ],
    )(_extract_kernel_body)
    cv, wv = extract(c, w, ct, wt, tailc, tailw)

    dot = functools.partial(
        pl.kernel,
        mesh=mesh,
        compiler_params=_COMPILER_PARAMS,
        out_type=jax.ShapeDtypeStruct((BATCH,), jnp.float32),
        scratch_types=[
            pltpu.VMEM((_BPW // 4, _ROWW), jnp.float32),
            pltpu.VMEM((_BPW // 4, _ROWW), jnp.float32),
            pltpu.VMEM((_BPW // 4, _ROWW), jnp.float32),
            pltpu.VMEM((_BPW // 4, _ROWW), jnp.float32),
            pltpu.VMEM((16, 16), jnp.float32),
            pltpu.VMEM((_BPW,), jnp.float32),
            pltpu.SemaphoreType.DMA,
            pltpu.SemaphoreType.DMA,
        ],
    )(_dot_body)
    return dot(cv, wv)


def kernel(c, w, c_embeds, w_embeds):
    tailc = c_embeds[_NFULL * _C:, :].T.reshape(-1)
    tailw = w_embeds[_NFULL * _C:, :].T.reshape(-1)
    return _sgns(c.astype(jnp.int32), w.astype(jnp.int32),
                 c_embeds.T, w_embeds.T, tailc, tailw)
